# baseline (speedup 1.0000x reference)
import jax
import jax.numpy as jnp
from jax.experimental import pallas as pl
from jax.experimental.pallas import tpu as pltpu

_BM = 6784
_OFF_BOX = 24
_OFF_OBJ = 72
_NPAD = 80


def _heads_kernel(x_ref, wc_ref, wb_ref, wo_ref, bc_ref, bb_ref, bo_ref,
                  cls_ref, box_ref, obj_ref, w_s, b_s):
    n_cls = cls_ref.shape[0]
    n_box = box_ref.shape[0]
    n_obj = obj_ref.shape[0]

    @pl.when(pl.program_id(0) == 0)
    def _init():
        w_s[...] = jnp.zeros_like(w_s)
        w_s[0:n_cls, :] = wc_ref[...]
        w_s[_OFF_BOX:_OFF_BOX + n_box, :] = wb_ref[...]
        w_s[_OFF_OBJ:_OFF_OBJ + n_obj, :] = wo_ref[...]
        b_s[...] = jnp.zeros_like(b_s)
        b_s[0:1, 0:n_cls] = bc_ref[...][None, :]
        b_s[0:1, _OFF_BOX:_OFF_BOX + n_box] = bb_ref[...][None, :]
        b_s[0:1, _OFF_OBJ:_OFF_OBJ + n_obj] = bo_ref[...][None, :]

    acc = jax.lax.dot_general(
        w_s[...], x_ref[...], (((1,), (1,)), ((), ())),
        preferred_element_type=jnp.float32)
    acc = acc + jnp.transpose(b_s[...])
    cls_ref[...] = acc[0:n_cls, :]
    box_ref[...] = acc[_OFF_BOX:_OFF_BOX + n_box, :]
    obj_ref[...] = acc[_OFF_OBJ:_OFF_OBJ + n_obj, :]


def kernel(x, W_cls, b_cls, W_box, b_box, W_obj, b_obj):
    M, K = x.shape
    n_cls = W_cls.shape[1]
    n_box = W_box.shape[1]
    n_obj = W_obj.shape[1]

    WcT, WbT, WoT = W_cls.T, W_box.T, W_obj.T

    grid = (pl.cdiv(M, _BM),)
    cls_t, box_t, obj_t = pl.pallas_call(
        _heads_kernel,
        grid=grid,
        in_specs=[
            pl.BlockSpec((_BM, K), lambda i: (i, 0)),
            pl.BlockSpec((n_cls, K), lambda i: (0, 0)),
            pl.BlockSpec((n_box, K), lambda i: (0, 0)),
            pl.BlockSpec((n_obj, K), lambda i: (0, 0)),
            pl.BlockSpec(memory_space=pltpu.VMEM),
            pl.BlockSpec(memory_space=pltpu.VMEM),
            pl.BlockSpec(memory_space=pltpu.VMEM),
        ],
        out_specs=[
            pl.BlockSpec((n_cls, _BM), lambda i: (0, i)),
            pl.BlockSpec((n_box, _BM), lambda i: (0, i)),
            pl.BlockSpec((n_obj, _BM), lambda i: (0, i)),
        ],
        out_shape=[
            jax.ShapeDtypeStruct((n_cls, M), x.dtype),
            jax.ShapeDtypeStruct((n_box, M), x.dtype),
            jax.ShapeDtypeStruct((n_obj, M), x.dtype),
        ],
        scratch_shapes=[
            pltpu.VMEM((_NPAD, K), jnp.float32),
            pltpu.VMEM((1, _NPAD), jnp.float32),
        ],
        compiler_params=pltpu.CompilerParams(
            dimension_semantics=("arbitrary",),
        ),
    )(x, WcT, WbT, WoT, b_cls, b_box, b_obj)
    return (cls_t.T, box_t.T, obj_t.T)


# speedup vs baseline: 1.1338x; 1.1338x over previous
import jax
import jax.numpy as jnp
from jax.experimental import pallas as pl
from jax.experimental.pallas import tpu as pltpu

_BM = 10240
_BH = _BM // 2
_OFF_BOX = 24
_OFF_OBJ = 72
_NPAD = 80


def _heads_kernel(xlo_ref, xhi_ref, wc_ref, wb_ref, wo_ref,
                  bc_ref, bb_ref, bo_ref,
                  cls_ref, box_ref, obj_ref, w_s, b_s):
    n_cls = cls_ref.shape[0]
    n_box = box_ref.shape[0]
    n_obj = obj_ref.shape[0]

    @pl.when(pl.program_id(0) == 0)
    def _init():
        w_s[...] = jnp.zeros_like(w_s)
        w_s[0:n_cls, :] = wc_ref[...]
        w_s[_OFF_BOX:_OFF_BOX + n_box, :] = wb_ref[...]
        w_s[_OFF_OBJ:_OFF_OBJ + n_obj, :] = wo_ref[...]
        b_s[...] = jnp.zeros_like(b_s)
        b_s[0:1, 0:n_cls] = bc_ref[...][None, :]
        b_s[0:1, _OFF_BOX:_OFF_BOX + n_box] = bb_ref[...][None, :]
        b_s[0:1, _OFF_OBJ:_OFF_OBJ + n_obj] = bo_ref[...][None, :]

    dims = (((1,), (1,)), ((), ()))
    b_col = jnp.transpose(b_s[...])
    acc_lo = jax.lax.dot_general(w_s[...], xlo_ref[...], dims,
                                 preferred_element_type=jnp.float32) + b_col
    acc_hi = jax.lax.dot_general(w_s[...], xhi_ref[...], dims,
                                 preferred_element_type=jnp.float32) + b_col
    for ref, lo, hi in (
        (cls_ref, 0, n_cls),
        (box_ref, _OFF_BOX, _OFF_BOX + n_box),
        (obj_ref, _OFF_OBJ, _OFF_OBJ + n_obj),
    ):
        ref[:, 0:_BH] = acc_lo[lo:hi, :]
        ref[:, _BH:_BM] = acc_hi[lo:hi, :]


def kernel(x, W_cls, b_cls, W_box, b_box, W_obj, b_obj):
    M, K = x.shape
    n_cls = W_cls.shape[1]
    n_box = W_box.shape[1]
    n_obj = W_obj.shape[1]

    WcT, WbT, WoT = W_cls.T, W_box.T, W_obj.T

    grid = (pl.cdiv(M, _BM),)
    cls_t, box_t, obj_t = pl.pallas_call(
        _heads_kernel,
        grid=grid,
        in_specs=[
            pl.BlockSpec((_BH, K), lambda i: (2 * i, 0)),
            pl.BlockSpec((_BH, K), lambda i: (2 * i + 1, 0)),
            pl.BlockSpec((n_cls, K), lambda i: (0, 0)),
            pl.BlockSpec((n_box, K), lambda i: (0, 0)),
            pl.BlockSpec((n_obj, K), lambda i: (0, 0)),
            pl.BlockSpec(memory_space=pltpu.VMEM),
            pl.BlockSpec(memory_space=pltpu.VMEM),
            pl.BlockSpec(memory_space=pltpu.VMEM),
        ],
        out_specs=[
            pl.BlockSpec((n_cls, _BM), lambda i: (0, i)),
            pl.BlockSpec((n_box, _BM), lambda i: (0, i)),
            pl.BlockSpec((n_obj, _BM), lambda i: (0, i)),
        ],
        out_shape=[
            jax.ShapeDtypeStruct((n_cls, M), x.dtype),
            jax.ShapeDtypeStruct((n_box, M), x.dtype),
            jax.ShapeDtypeStruct((n_obj, M), x.dtype),
        ],
        scratch_shapes=[
            pltpu.VMEM((_NPAD, K), jnp.float32),
            pltpu.VMEM((1, _NPAD), jnp.float32),
        ],
        compiler_params=pltpu.CompilerParams(
            dimension_semantics=("arbitrary",),
        ),
    )(x, x, WcT, WbT, WoT, b_cls, b_box, b_obj)
    return (cls_t.T, box_t.T, obj_t.T)
